# baseline (speedup 1.0000x reference)
import functools

import jax
import jax.numpy as jnp
from jax import lax
from jax.experimental import pallas as pl
from jax.experimental.pallas import tpu as pltpu
from jax.experimental.pallas import tpu_sc as plsc

N = 100000
VA = 1000000
VR = 100000
D_ATOM = 64
D_RES = 32
D_OUT = 112

NC = 2
NS = 16
NW = NC * NS

CHUNK = 224
NCHUNK = 14
BPW = CHUNK * NCHUNK
NP = NW * BPW

VB = 8192
NB = 8192


def _eye(n):
    r = lax.broadcasted_iota(jnp.int32, (n, n), 0)
    c = lax.broadcasted_iota(jnp.int32, (n, n), 1)
    return jnp.where(r == c, 1.0, 0.0).astype(jnp.float32)


def _mxu_t(x):
    ident = _eye(x.shape[0])
    return lax.dot_general(x, ident, (((0,), (0,)), ((), ())),
                           preferred_element_type=jnp.float32)


def _relayout_atom(wt):
    def body(in_ref, out_ref):
        y = _mxu_t(in_ref[...])
        out_ref[...] = jnp.concatenate([y, y], axis=1)

    grid = pl.cdiv(VA, VB)
    return pl.pallas_call(
        body,
        grid=(grid,),
        in_specs=[pl.BlockSpec((64, VB), lambda b: (0, b))],
        out_specs=pl.BlockSpec((VB, 128), lambda b: (b, 0)),
        out_shape=jax.ShapeDtypeStruct((VA, 128), jnp.float32),
        compiler_params=pltpu.CompilerParams(
            dimension_semantics=("arbitrary",),
            vmem_limit_bytes=100 * 1024 * 1024),
    )(wt)


def _relayout_res(wt):
    def body(in_ref, out_ref):
        y = _mxu_t(in_ref[...])
        out_ref[...] = jnp.concatenate([y, y, y, y], axis=1)

    grid = pl.cdiv(VR, VB)
    return pl.pallas_call(
        body,
        grid=(grid,),
        in_specs=[pl.BlockSpec((32, VB), lambda b: (0, b))],
        out_specs=pl.BlockSpec((VB, 128), lambda b: (b, 0)),
        out_shape=jax.ShapeDtypeStruct((VR, 128), jnp.float32),
        compiler_params=pltpu.CompilerParams(
            dimension_semantics=("arbitrary",),
            vmem_limit_bytes=100 * 1024 * 1024),
    )(wt)


def _finalize(out_sc, ch_t):
    def body(x_ref, c_ref, out_ref):
        y = _mxu_t(x_ref[...])
        out_ref[0:96, :] = y[0:96, :]
        out_ref[96:112, :] = c_ref[...]

    grid = pl.cdiv(N, NB)
    return pl.pallas_call(
        body,
        grid=(grid,),
        in_specs=[
            pl.BlockSpec((NB, 128), lambda b: (b, 0)),
            pl.BlockSpec((16, NB), lambda b: (0, b)),
        ],
        out_specs=pl.BlockSpec((D_OUT, NB), lambda b: (0, b)),
        out_shape=jax.ShapeDtypeStruct((D_OUT, N), jnp.float32),
        compiler_params=pltpu.CompilerParams(
            dimension_semantics=("arbitrary",),
            vmem_limit_bytes=100 * 1024 * 1024),
    )(out_sc, ch_t)


def _make_sc_kernel():
    mesh = plsc.VectorSubcoreMesh(core_axis_name="c", subcore_axis_name="s")

    @functools.partial(
        pl.kernel,
        mesh=mesh,
        out_type=jax.ShapeDtypeStruct((NP, 128), jnp.float32),
        compiler_params=pltpu.CompilerParams(needs_layout_passes=False),
        scratch_types=[
            pltpu.VMEM((CHUNK,), jnp.int32),
            pltpu.VMEM((CHUNK,), jnp.int32),
            pltpu.VMEM((CHUNK,), jnp.int32),
            pltpu.VMEM((CHUNK,), jnp.int32),
            pltpu.VMEM((CHUNK, 128), jnp.float32),
            pltpu.VMEM((CHUNK, 128), jnp.float32),
            pltpu.VMEM((CHUNK, 128), jnp.float32),
            pltpu.VMEM((CHUNK, 128), jnp.float32),
            pltpu.SemaphoreType.DMA,
            pltpu.SemaphoreType.DMA,
            pltpu.SemaphoreType.DMA,
            pltpu.SemaphoreType.DMA,
        ],
    )
    def emb_kernel(idx_a, idx_r, wa, wr, out,
                   idxa0, idxa1, idxr0, idxr1,
                   stage0, stage1, gr0, gr1, sa0, sa1, sr0, sr1):
        wid = lax.axis_index("s") * NC + lax.axis_index("c")
        base = wid * BPW
        idxa = (idxa0, idxa1)
        idxr = (idxr0, idxr1)
        stage = (stage0, stage1)
        gr = (gr0, gr1)
        sem_a = (sa0, sa1)
        sem_r = (sr0, sr1)

        def fetch(ci, b):
            start = base + ci * CHUNK
            pltpu.sync_copy(idx_a.at[pl.ds(start, CHUNK)], idxa[b])
            pltpu.sync_copy(idx_r.at[pl.ds(start, CHUNK)], idxr[b])
            pltpu.async_copy(wa.at[idxa[b]], stage[b], sem_a[b])
            pltpu.async_copy(wr.at[idxr[b]], gr[b], sem_r[b])

        fetch(0, 0)

        def pair_body(g, carry):
            for b in (0, 1):
                ci = 2 * g + b

                @pl.when(ci + 1 < NCHUNK)
                def _prefetch():
                    fetch(ci + 1, 1 - b)

                pltpu.make_async_copy(
                    wa.at[idxa[b]], stage[b], sem_a[b]).wait()
                pltpu.make_async_copy(
                    wr.at[idxr[b]], gr[b], sem_r[b]).wait()

                def node_body(i, carry2):
                    for l in range(8):
                        j = i * 8 + l
                        for m in range(2):
                            stage[b][j, pl.ds(64 + 16 * m, 16)] = (
                                gr[b][j, pl.ds(16 * m, 16)])
                    return carry2

                lax.fori_loop(0, CHUNK // 8, node_body, 0)
                pltpu.sync_copy(stage[b],
                                out.at[pl.ds(base + ci * CHUNK, CHUNK)])
            return carry

        lax.fori_loop(0, NCHUNK // 2, pair_body, 0)

    return emb_kernel


_SC_EMB = _make_sc_kernel()


def kernel(atom_type, residue_type, charge, W_atom, W_res):
    idx_a = atom_type.reshape(-1).astype(jnp.int32)
    idx_r = residue_type.reshape(-1).astype(jnp.int32)
    pad = (0, NP - N)
    idx_a_p = jnp.pad(idx_a, pad)
    idx_r_p = jnp.pad(idx_r, pad)
    wa = _relayout_atom(W_atom.T)
    wr = _relayout_res(W_res.T)
    out = _SC_EMB(idx_a_p, idx_r_p, wa, wr)
    return _finalize(out, charge.T).T


# speedup vs baseline: 1.4864x; 1.4864x over previous
import functools

import jax
import jax.numpy as jnp
from jax import lax
from jax.experimental import pallas as pl
from jax.experimental.pallas import tpu as pltpu
from jax.experimental.pallas import tpu_sc as plsc

N = 100000
VA = 1000000
VR = 100000
D_ATOM = 64
D_RES = 32
D_OUT = 112

NC = 2
NS = 16
NW = NC * NS

CHUNK = 224
NCHUNK = 14
BPW = CHUNK * NCHUNK
NP = NW * BPW

VB = 8192
NB = 8192


def _relayout_atom(wt):
    def body(in_ref, out_ref):
        y = in_ref[...].T
        out_ref[...] = jnp.concatenate([y, y], axis=1)

    grid = pl.cdiv(VA, VB)
    return pl.pallas_call(
        body,
        grid=(grid,),
        in_specs=[pl.BlockSpec((64, VB), lambda b: (0, b))],
        out_specs=pl.BlockSpec((VB, 128), lambda b: (b, 0)),
        out_shape=jax.ShapeDtypeStruct((VA, 128), jnp.float32),
        compiler_params=pltpu.CompilerParams(
            dimension_semantics=("arbitrary",),
            vmem_limit_bytes=100 * 1024 * 1024),
    )(wt)


VRP = 102400
RPW = VRP // NW
RBLK = 640


def _make_sc_res_relayout():
    mesh = plsc.VectorSubcoreMesh(core_axis_name="c", subcore_axis_name="s")

    @functools.partial(
        pl.kernel,
        mesh=mesh,
        out_type=jax.ShapeDtypeStruct((VRP, 128), jnp.float32),
        compiler_params=pltpu.CompilerParams(needs_layout_passes=False),
        scratch_types=[
            pltpu.VMEM((32, RBLK), jnp.float32),
            pltpu.VMEM((RBLK, 128), jnp.float32),
        ],
    )
    def res_kernel(wt, wr, wtv, st):
        wid = lax.axis_index("s") * NC + lax.axis_index("c")
        vbase = wid * RPW

        def blk_body(hb, carry):
            v0 = vbase + hb * RBLK
            pltpu.sync_copy(wt.at[:, pl.ds(v0, RBLK)], wtv)

            def v_body(v, carry2):
                cols = jnp.full((16,), v, jnp.int32)
                g0 = plsc.load_gather(
                    wtv, [lax.iota(jnp.int32, 16), cols])
                g1 = plsc.load_gather(
                    wtv, [lax.iota(jnp.int32, 16) + 16, cols])
                st[v, pl.ds(0, 16)] = g0
                st[v, pl.ds(16, 16)] = g1
                return carry2

            lax.fori_loop(0, RBLK, v_body, 0)
            pltpu.sync_copy(st, wr.at[pl.ds(v0, RBLK)])
            return carry

        lax.fori_loop(0, RPW // RBLK, blk_body, 0)

    return res_kernel


_SC_RES = _make_sc_res_relayout()


def _finalize(out_sc, ch_t):
    def body(x_ref, c_ref, out_ref):
        y = x_ref[...].T
        out_ref[0:96, :] = y[0:96, :]
        out_ref[96:112, :] = c_ref[...]

    grid = pl.cdiv(N, NB)
    return pl.pallas_call(
        body,
        grid=(grid,),
        in_specs=[
            pl.BlockSpec((NB, 128), lambda b: (b, 0)),
            pl.BlockSpec((16, NB), lambda b: (0, b)),
        ],
        out_specs=pl.BlockSpec((D_OUT, NB), lambda b: (0, b)),
        out_shape=jax.ShapeDtypeStruct((D_OUT, N), jnp.float32),
        compiler_params=pltpu.CompilerParams(
            dimension_semantics=("arbitrary",),
            vmem_limit_bytes=100 * 1024 * 1024),
    )(out_sc, ch_t)


def _make_sc_kernel():
    mesh = plsc.VectorSubcoreMesh(core_axis_name="c", subcore_axis_name="s")

    @functools.partial(
        pl.kernel,
        mesh=mesh,
        out_type=jax.ShapeDtypeStruct((NP, 128), jnp.float32),
        compiler_params=pltpu.CompilerParams(needs_layout_passes=False),
        scratch_types=[
            pltpu.VMEM((CHUNK,), jnp.int32),
            pltpu.VMEM((CHUNK,), jnp.int32),
            pltpu.VMEM((CHUNK,), jnp.int32),
            pltpu.VMEM((CHUNK,), jnp.int32),
            pltpu.VMEM((CHUNK, 128), jnp.float32),
            pltpu.VMEM((CHUNK, 128), jnp.float32),
            pltpu.VMEM((CHUNK, 128), jnp.float32),
            pltpu.VMEM((CHUNK, 128), jnp.float32),
            pltpu.SemaphoreType.DMA,
            pltpu.SemaphoreType.DMA,
            pltpu.SemaphoreType.DMA,
            pltpu.SemaphoreType.DMA,
        ],
    )
    def emb_kernel(idx_a, idx_r, wa, wr, out,
                   idxa0, idxa1, idxr0, idxr1,
                   stage0, stage1, gr0, gr1, sa0, sa1, sr0, sr1):
        wid = lax.axis_index("s") * NC + lax.axis_index("c")
        base = wid * BPW
        idxa = (idxa0, idxa1)
        idxr = (idxr0, idxr1)
        stage = (stage0, stage1)
        gr = (gr0, gr1)
        sem_a = (sa0, sa1)
        sem_r = (sr0, sr1)

        def fetch(ci, b):
            start = base + ci * CHUNK
            pltpu.sync_copy(idx_a.at[pl.ds(start, CHUNK)], idxa[b])
            pltpu.sync_copy(idx_r.at[pl.ds(start, CHUNK)], idxr[b])
            pltpu.async_copy(wa.at[idxa[b]], stage[b], sem_a[b])
            pltpu.async_copy(wr.at[idxr[b]], gr[b], sem_r[b])

        fetch(0, 0)

        def pair_body(g, carry):
            for b in (0, 1):
                ci = 2 * g + b

                @pl.when(ci + 1 < NCHUNK)
                def _prefetch():
                    fetch(ci + 1, 1 - b)

                pltpu.make_async_copy(
                    wa.at[idxa[b]], stage[b], sem_a[b]).wait()
                pltpu.make_async_copy(
                    wr.at[idxr[b]], gr[b], sem_r[b]).wait()

                def node_body(i, carry2):
                    for l in range(8):
                        j = i * 8 + l
                        for m in range(2):
                            stage[b][j, pl.ds(64 + 16 * m, 16)] = (
                                gr[b][j, pl.ds(16 * m, 16)])
                    return carry2

                lax.fori_loop(0, CHUNK // 8, node_body, 0)
                pltpu.sync_copy(stage[b],
                                out.at[pl.ds(base + ci * CHUNK, CHUNK)])
            return carry

        lax.fori_loop(0, NCHUNK // 2, pair_body, 0)

    return emb_kernel


_SC_EMB = _make_sc_kernel()


def kernel(atom_type, residue_type, charge, W_atom, W_res):
    idx_a = atom_type.reshape(-1).astype(jnp.int32)
    idx_r = residue_type.reshape(-1).astype(jnp.int32)
    pad = (0, NP - N)
    idx_a_p = jnp.pad(idx_a, pad)
    idx_r_p = jnp.pad(idx_r, pad)
    wt_res = jnp.pad(W_res.T, ((0, 0), (0, VRP - VR)))
    wr = _SC_RES(wt_res)
    wa = _relayout_atom(W_atom.T)
    out = _SC_EMB(idx_a_p, idx_r_p, wa, wr)
    return _finalize(out, charge.T).T


# speedup vs baseline: 1.8382x; 1.2367x over previous
import functools

import jax
import jax.numpy as jnp
from jax import lax
from jax.experimental import pallas as pl
from jax.experimental.pallas import tpu as pltpu
from jax.experimental.pallas import tpu_sc as plsc

N = 100000
VA = 1000000
VR = 100000
D_ATOM = 64
D_RES = 32
D_OUT = 112

NC = 2
NS = 16
NW = NC * NS

CHUNK = 224
NCHUNK = 14
BPW = CHUNK * NCHUNK
NP = NW * BPW

VB = 8192
NB = 8192
HALF = 499712
NBLK_A = 62
WAR = NBLK_A * VB


def _relayout_atom(wt):
    def body(in1_ref, in2_ref, out_ref):
        y1 = in1_ref[...].T
        y2 = in2_ref[...].T
        out_ref[...] = jnp.concatenate([y1, y2], axis=1)

    return pl.pallas_call(
        body,
        grid=(NBLK_A,),
        in_specs=[
            pl.BlockSpec((64, VB), lambda b: (0, jnp.where(b < 61, b, 122))),
            pl.BlockSpec((64, VB),
                         lambda b: (0, jnp.where(b < 61, b + 61, 122))),
        ],
        out_specs=pl.BlockSpec((VB, 128), lambda b: (b, 0)),
        out_shape=jax.ShapeDtypeStruct((WAR, 128), jnp.float32),
        compiler_params=pltpu.CompilerParams(
            dimension_semantics=("arbitrary",),
            vmem_limit_bytes=100 * 1024 * 1024),
    )(wt, wt)


VRP = 102400
RPW = VRP // NW
RBLK = 640


def _make_sc_res_relayout():
    mesh = plsc.VectorSubcoreMesh(core_axis_name="c", subcore_axis_name="s")

    @functools.partial(
        pl.kernel,
        mesh=mesh,
        out_type=jax.ShapeDtypeStruct((VRP, 128), jnp.float32),
        compiler_params=pltpu.CompilerParams(needs_layout_passes=False),
        scratch_types=[
            pltpu.VMEM((32, RBLK), jnp.float32),
            pltpu.VMEM((RBLK, 128), jnp.float32),
        ],
    )
    def res_kernel(wt, wr, wtv, st):
        wid = lax.axis_index("s") * NC + lax.axis_index("c")
        vbase = wid * RPW

        def blk_body(hb, carry):
            v0 = vbase + hb * RBLK
            pltpu.sync_copy(wt.at[:, pl.ds(v0, RBLK)], wtv)

            def v_body(v, carry2):
                cols = jnp.full((16,), v, jnp.int32)
                g0 = plsc.load_gather(
                    wtv, [lax.iota(jnp.int32, 16), cols])
                g1 = plsc.load_gather(
                    wtv, [lax.iota(jnp.int32, 16) + 16, cols])
                st[v, pl.ds(0, 16)] = g0
                st[v, pl.ds(16, 16)] = g1
                return carry2

            lax.fori_loop(0, RBLK, v_body, 0)
            pltpu.sync_copy(st, wr.at[pl.ds(v0, RBLK)])
            return carry

        lax.fori_loop(0, RPW // RBLK, blk_body, 0)

    return res_kernel


_SC_RES = _make_sc_res_relayout()


def _finalize(out_sc, ch_t):
    def body(x_ref, c_ref, out_ref):
        y = x_ref[...].T
        out_ref[0:96, :] = y[0:96, :]
        out_ref[96:112, :] = c_ref[...]

    grid = pl.cdiv(N, NB)
    return pl.pallas_call(
        body,
        grid=(grid,),
        in_specs=[
            pl.BlockSpec((NB, 128), lambda b: (b, 0)),
            pl.BlockSpec((16, NB), lambda b: (0, b)),
        ],
        out_specs=pl.BlockSpec((D_OUT, NB), lambda b: (0, b)),
        out_shape=jax.ShapeDtypeStruct((D_OUT, N), jnp.float32),
        compiler_params=pltpu.CompilerParams(
            dimension_semantics=("arbitrary",),
            vmem_limit_bytes=100 * 1024 * 1024),
    )(out_sc, ch_t)


def _make_sc_kernel():
    mesh = plsc.VectorSubcoreMesh(core_axis_name="c", subcore_axis_name="s")

    @functools.partial(
        pl.kernel,
        mesh=mesh,
        out_type=jax.ShapeDtypeStruct((NP, 128), jnp.float32),
        compiler_params=pltpu.CompilerParams(needs_layout_passes=False),
        scratch_types=[
            pltpu.VMEM((CHUNK,), jnp.int32),
            pltpu.VMEM((CHUNK,), jnp.int32),
            pltpu.VMEM((CHUNK,), jnp.int32),
            pltpu.VMEM((CHUNK,), jnp.int32),
            pltpu.VMEM((CHUNK,), jnp.int32),
            pltpu.VMEM((CHUNK,), jnp.int32),
            pltpu.VMEM((CHUNK, 128), jnp.float32),
            pltpu.VMEM((CHUNK, 128), jnp.float32),
            pltpu.VMEM((CHUNK, 128), jnp.float32),
            pltpu.VMEM((CHUNK, 128), jnp.float32),
            pltpu.SemaphoreType.DMA,
            pltpu.SemaphoreType.DMA,
            pltpu.SemaphoreType.DMA,
            pltpu.SemaphoreType.DMA,
        ],
    )
    def emb_kernel(idx_a, idx_r, wa, wr, out,
                   pv0, pv1, offv0, offv1, idxr0, idxr1,
                   stage0, stage1, gr0, gr1, sa0, sa1, sr0, sr1):
        wid = lax.axis_index("s") * NC + lax.axis_index("c")
        base = wid * BPW
        pv = (pv0, pv1)
        offv = (offv0, offv1)
        idxr = (idxr0, idxr1)
        stage = (stage0, stage1)
        gr = (gr0, gr1)
        sem_a = (sa0, sa1)
        sem_r = (sr0, sr1)

        def fetch(ci, b):
            start = base + ci * CHUNK
            pltpu.sync_copy(idx_a.at[pl.ds(start, CHUNK)], pv[b])
            pltpu.sync_copy(idx_r.at[pl.ds(start, CHUNK)], idxr[b])

            def map_body(g2, carry3):
                j0 = g2 * 16
                iv = pv[b][pl.ds(j0, 16)]
                hi = iv >= HALF
                pv[b][pl.ds(j0, 16)] = iv - jnp.where(hi, HALF, 0)
                offv[b][pl.ds(j0, 16)] = jnp.where(
                    hi & (iv < 2 * HALF), 64, 0)
                return carry3

            lax.fori_loop(0, CHUNK // 16, map_body, 0)
            pltpu.async_copy(wa.at[pv[b]], stage[b], sem_a[b])
            pltpu.async_copy(wr.at[idxr[b]], gr[b], sem_r[b])

        fetch(0, 0)

        def pair_body(g, carry):
            for b in (0, 1):
                ci = 2 * g + b

                @pl.when(ci + 1 < NCHUNK)
                def _prefetch():
                    fetch(ci + 1, 1 - b)

                pltpu.make_async_copy(
                    wa.at[pv[b]], stage[b], sem_a[b]).wait()
                pltpu.make_async_copy(
                    wr.at[idxr[b]], gr[b], sem_r[b]).wait()

                def node_body(i, carry2):
                    j0 = i * 16
                    ov = offv[b][pl.ds(j0, 16)]
                    for l in range(16):
                        j = j0 + l

                        @pl.when(ov[l] != 0)
                        def _shift_atom():
                            for k in range(4):
                                stage[b][j, pl.ds(16 * k, 16)] = (
                                    stage[b][j, pl.ds(64 + 16 * k, 16)])

                        for m in range(2):
                            stage[b][j, pl.ds(64 + 16 * m, 16)] = (
                                gr[b][j, pl.ds(16 * m, 16)])
                    return carry2

                lax.fori_loop(0, CHUNK // 16, node_body, 0)
                pltpu.sync_copy(stage[b],
                                out.at[pl.ds(base + ci * CHUNK, CHUNK)])
            return carry

        lax.fori_loop(0, NCHUNK // 2, pair_body, 0)

    return emb_kernel


_SC_EMB = _make_sc_kernel()


def kernel(atom_type, residue_type, charge, W_atom, W_res):
    idx_a = atom_type.reshape(-1).astype(jnp.int32)
    idx_r = residue_type.reshape(-1).astype(jnp.int32)
    pad = (0, NP - N)
    idx_a_p = jnp.pad(idx_a, pad)
    idx_r_p = jnp.pad(idx_r, pad)
    wt_res = jnp.pad(W_res.T, ((0, 0), (0, VRP - VR)))
    wr = _SC_RES(wt_res)
    wa = _relayout_atom(W_atom.T)
    out = _SC_EMB(idx_a_p, idx_r_p, wa, wr)
    return _finalize(out, charge.T).T


# speedup vs baseline: 1.8724x; 1.0186x over previous
import functools

import jax
import jax.numpy as jnp
from jax import lax
from jax.experimental import pallas as pl
from jax.experimental.pallas import tpu as pltpu
from jax.experimental.pallas import tpu_sc as plsc

N = 100000
VA = 1000000
VR = 100000
D_ATOM = 64
D_RES = 32
D_OUT = 112

NC = 2
NS = 16
NW = NC * NS

CHUNK = 224
NCHUNK = 14
BPW = CHUNK * NCHUNK
NP = NW * BPW

VB = 16384
NB = 8192
HALF = 491520
NBLK_A = 32
WAR = NBLK_A * VB


def _relayout_atom(wt):
    def body(in1_ref, in2_ref, out_ref):
        y1 = in1_ref[...].T
        y2 = in2_ref[...].T
        out_ref[...] = jnp.concatenate([y1, y2], axis=1)

    return pl.pallas_call(
        body,
        grid=(NBLK_A,),
        in_specs=[
            pl.BlockSpec((64, VB), lambda b: (0, jnp.where(b < 30, b, b + 30))),
            pl.BlockSpec((64, VB),
                         lambda b: (0, jnp.where(b < 30, b + 30, b + 30))),
        ],
        out_specs=pl.BlockSpec((VB, 128), lambda b: (b, 0)),
        out_shape=jax.ShapeDtypeStruct((WAR, 128), jnp.float32),
        compiler_params=pltpu.CompilerParams(
            dimension_semantics=("arbitrary",),
            vmem_limit_bytes=100 * 1024 * 1024),
    )(wt, wt)


VRP = 102400
RPW = VRP // NW
RBLK = 640


def _make_sc_res_relayout():
    mesh = plsc.VectorSubcoreMesh(core_axis_name="c", subcore_axis_name="s")

    @functools.partial(
        pl.kernel,
        mesh=mesh,
        out_type=jax.ShapeDtypeStruct((VRP, 128), jnp.float32),
        compiler_params=pltpu.CompilerParams(needs_layout_passes=False),
        scratch_types=[
            pltpu.VMEM((32, RBLK), jnp.float32),
            pltpu.VMEM((RBLK, 128), jnp.float32),
        ],
    )
    def res_kernel(wt, wr, wtv, st):
        wid = lax.axis_index("s") * NC + lax.axis_index("c")
        vbase = wid * RPW

        def blk_body(hb, carry):
            v0 = vbase + hb * RBLK
            pltpu.sync_copy(wt.at[:, pl.ds(v0, RBLK)], wtv)

            def v_body(v, carry2):
                cols = jnp.full((16,), v, jnp.int32)
                g0 = plsc.load_gather(
                    wtv, [lax.iota(jnp.int32, 16), cols])
                g1 = plsc.load_gather(
                    wtv, [lax.iota(jnp.int32, 16) + 16, cols])
                st[v, pl.ds(0, 16)] = g0
                st[v, pl.ds(16, 16)] = g1
                return carry2

            lax.fori_loop(0, RBLK, v_body, 0)
            pltpu.sync_copy(st, wr.at[pl.ds(v0, RBLK)])
            return carry

        lax.fori_loop(0, RPW // RBLK, blk_body, 0)

    return res_kernel


_SC_RES = _make_sc_res_relayout()


def _finalize(out_sc, ch_t):
    def body(x_ref, c_ref, out_ref):
        y = x_ref[...].T
        out_ref[0:96, :] = y[0:96, :]
        out_ref[96:112, :] = c_ref[...]

    grid = pl.cdiv(N, NB)
    return pl.pallas_call(
        body,
        grid=(grid,),
        in_specs=[
            pl.BlockSpec((NB, 128), lambda b: (b, 0)),
            pl.BlockSpec((16, NB), lambda b: (0, b)),
        ],
        out_specs=pl.BlockSpec((D_OUT, NB), lambda b: (0, b)),
        out_shape=jax.ShapeDtypeStruct((D_OUT, N), jnp.float32),
        compiler_params=pltpu.CompilerParams(
            dimension_semantics=("arbitrary",),
            vmem_limit_bytes=100 * 1024 * 1024),
    )(out_sc, ch_t)


def _make_sc_kernel():
    mesh = plsc.VectorSubcoreMesh(core_axis_name="c", subcore_axis_name="s")

    @functools.partial(
        pl.kernel,
        mesh=mesh,
        out_type=jax.ShapeDtypeStruct((NP, 128), jnp.float32),
        compiler_params=pltpu.CompilerParams(needs_layout_passes=False),
        scratch_types=[
            pltpu.VMEM((CHUNK,), jnp.int32),
            pltpu.VMEM((CHUNK,), jnp.int32),
            pltpu.VMEM((CHUNK,), jnp.int32),
            pltpu.VMEM((CHUNK,), jnp.int32),
            pltpu.VMEM((CHUNK,), jnp.int32),
            pltpu.VMEM((CHUNK,), jnp.int32),
            pltpu.VMEM((CHUNK, 128), jnp.float32),
            pltpu.VMEM((CHUNK, 128), jnp.float32),
            pltpu.VMEM((CHUNK, 128), jnp.float32),
            pltpu.VMEM((CHUNK, 128), jnp.float32),
            pltpu.SemaphoreType.DMA,
            pltpu.SemaphoreType.DMA,
            pltpu.SemaphoreType.DMA,
            pltpu.SemaphoreType.DMA,
        ],
    )
    def emb_kernel(idx_a, idx_r, wa, wr, out,
                   pv0, pv1, offv0, offv1, idxr0, idxr1,
                   stage0, stage1, gr0, gr1, sa0, sa1, sr0, sr1):
        wid = lax.axis_index("s") * NC + lax.axis_index("c")
        base = wid * BPW
        pv = (pv0, pv1)
        offv = (offv0, offv1)
        idxr = (idxr0, idxr1)
        stage = (stage0, stage1)
        gr = (gr0, gr1)
        sem_a = (sa0, sa1)
        sem_r = (sr0, sr1)

        def fetch(ci, b):
            start = base + ci * CHUNK
            pltpu.sync_copy(idx_a.at[pl.ds(start, CHUNK)], pv[b])
            pltpu.sync_copy(idx_r.at[pl.ds(start, CHUNK)], idxr[b])

            def map_body(g2, carry3):
                j0 = g2 * 16
                iv = pv[b][pl.ds(j0, 16)]
                hi = iv >= HALF
                pv[b][pl.ds(j0, 16)] = iv - jnp.where(hi, HALF, 0)
                offv[b][pl.ds(j0, 16)] = jnp.where(
                    hi & (iv < 2 * HALF), 64, 0)
                return carry3

            lax.fori_loop(0, CHUNK // 16, map_body, 0)
            pltpu.async_copy(wa.at[pv[b]], stage[b], sem_a[b])
            pltpu.async_copy(wr.at[idxr[b]], gr[b], sem_r[b])

        fetch(0, 0)

        def pair_body(g, carry):
            for b in (0, 1):
                ci = 2 * g + b

                @pl.when(ci + 1 < NCHUNK)
                def _prefetch():
                    fetch(ci + 1, 1 - b)

                pltpu.make_async_copy(
                    wa.at[pv[b]], stage[b], sem_a[b]).wait()
                pltpu.make_async_copy(
                    wr.at[idxr[b]], gr[b], sem_r[b]).wait()

                def node_body(i, carry2):
                    j0 = i * 16
                    ov = offv[b][pl.ds(j0, 16)]
                    for l in range(16):
                        j = j0 + l

                        @pl.when(ov[l] != 0)
                        def _shift_atom():
                            for k in range(4):
                                stage[b][j, pl.ds(16 * k, 16)] = (
                                    stage[b][j, pl.ds(64 + 16 * k, 16)])

                        for m in range(2):
                            stage[b][j, pl.ds(64 + 16 * m, 16)] = (
                                gr[b][j, pl.ds(16 * m, 16)])
                    return carry2

                lax.fori_loop(0, CHUNK // 16, node_body, 0)
                pltpu.sync_copy(stage[b],
                                out.at[pl.ds(base + ci * CHUNK, CHUNK)])
            return carry

        lax.fori_loop(0, NCHUNK // 2, pair_body, 0)

    return emb_kernel


_SC_EMB = _make_sc_kernel()


def kernel(atom_type, residue_type, charge, W_atom, W_res):
    idx_a = atom_type.reshape(-1).astype(jnp.int32)
    idx_r = residue_type.reshape(-1).astype(jnp.int32)
    pad = (0, NP - N)
    idx_a_p = jnp.pad(idx_a, pad)
    idx_r_p = jnp.pad(idx_r, pad)
    wt_res = jnp.pad(W_res.T, ((0, 0), (0, VRP - VR)))
    wr = _SC_RES(wt_res)
    wa = _relayout_atom(W_atom.T)
    out = _SC_EMB(idx_a_p, idx_r_p, wa, wr)
    return _finalize(out, charge.T).T


# speedup vs baseline: 1.8819x; 1.0051x over previous
import functools

import jax
import jax.numpy as jnp
from jax import lax
from jax.experimental import pallas as pl
from jax.experimental.pallas import tpu as pltpu
from jax.experimental.pallas import tpu_sc as plsc

N = 100000
VA = 1000000
VR = 100000
D_ATOM = 64
D_RES = 32
D_OUT = 112

NC = 2
NS = 16
NW = NC * NS

CHUNK = 224
NCHUNK = 14
BPW = CHUNK * NCHUNK
NP = NW * BPW

VB = 16384
NB = 16384
HALF = 491520
NBLK_A = 32
WAR = NBLK_A * VB


def _relayout_atom(wt):
    def body(in1_ref, in2_ref, out_ref):
        y1 = in1_ref[...].T
        y2 = in2_ref[...].T
        out_ref[...] = jnp.concatenate([y1, y2], axis=1)

    return pl.pallas_call(
        body,
        grid=(NBLK_A,),
        in_specs=[
            pl.BlockSpec((64, VB), lambda b: (0, jnp.where(b < 30, b, b + 30))),
            pl.BlockSpec((64, VB),
                         lambda b: (0, jnp.where(b < 30, b + 30, b + 30))),
        ],
        out_specs=pl.BlockSpec((VB, 128), lambda b: (b, 0)),
        out_shape=jax.ShapeDtypeStruct((WAR, 128), jnp.float32),
        compiler_params=pltpu.CompilerParams(
            dimension_semantics=("arbitrary",),
            vmem_limit_bytes=100 * 1024 * 1024),
    )(wt, wt)


VRP = 102400
RPW = VRP // NW
RBLK = 640


def _make_sc_res_relayout():
    mesh = plsc.VectorSubcoreMesh(core_axis_name="c", subcore_axis_name="s")

    @functools.partial(
        pl.kernel,
        mesh=mesh,
        out_type=jax.ShapeDtypeStruct((VRP, 128), jnp.float32),
        compiler_params=pltpu.CompilerParams(needs_layout_passes=False),
        scratch_types=[
            pltpu.VMEM((32, RBLK), jnp.float32),
            pltpu.VMEM((RBLK, 128), jnp.float32),
        ],
    )
    def res_kernel(wt, wr, wtv, st):
        wid = lax.axis_index("s") * NC + lax.axis_index("c")
        vbase = wid * RPW

        def blk_body(hb, carry):
            v0 = vbase + hb * RBLK
            pltpu.sync_copy(wt.at[:, pl.ds(v0, RBLK)], wtv)

            def v_body(v, carry2):
                cols = jnp.full((16,), v, jnp.int32)
                g0 = plsc.load_gather(
                    wtv, [lax.iota(jnp.int32, 16), cols])
                g1 = plsc.load_gather(
                    wtv, [lax.iota(jnp.int32, 16) + 16, cols])
                st[v, pl.ds(0, 16)] = g0
                st[v, pl.ds(16, 16)] = g1
                return carry2

            lax.fori_loop(0, RBLK, v_body, 0)
            pltpu.sync_copy(st, wr.at[pl.ds(v0, RBLK)])
            return carry

        lax.fori_loop(0, RPW // RBLK, blk_body, 0)

    return res_kernel


_SC_RES = _make_sc_res_relayout()


def _finalize(out_sc, ch_t):
    def body(x_ref, c_ref, out_ref):
        y = x_ref[...].T
        out_ref[0:96, :] = y[0:96, :]
        out_ref[96:112, :] = c_ref[...]

    grid = pl.cdiv(N, NB)
    return pl.pallas_call(
        body,
        grid=(grid,),
        in_specs=[
            pl.BlockSpec((NB, 128), lambda b: (b, 0)),
            pl.BlockSpec((16, NB), lambda b: (0, b)),
        ],
        out_specs=pl.BlockSpec((D_OUT, NB), lambda b: (0, b)),
        out_shape=jax.ShapeDtypeStruct((D_OUT, N), jnp.float32),
        compiler_params=pltpu.CompilerParams(
            dimension_semantics=("arbitrary",),
            vmem_limit_bytes=100 * 1024 * 1024),
    )(out_sc, ch_t)


def _make_sc_kernel():
    mesh = plsc.VectorSubcoreMesh(core_axis_name="c", subcore_axis_name="s")

    @functools.partial(
        pl.kernel,
        mesh=mesh,
        out_type=jax.ShapeDtypeStruct((NP, 128), jnp.float32),
        compiler_params=pltpu.CompilerParams(needs_layout_passes=False),
        scratch_types=[
            pltpu.VMEM((CHUNK,), jnp.int32),
            pltpu.VMEM((CHUNK,), jnp.int32),
            pltpu.VMEM((CHUNK,), jnp.int32),
            pltpu.VMEM((CHUNK,), jnp.int32),
            pltpu.VMEM((CHUNK,), jnp.int32),
            pltpu.VMEM((CHUNK,), jnp.int32),
            pltpu.VMEM((CHUNK, 128), jnp.float32),
            pltpu.VMEM((CHUNK, 128), jnp.float32),
            pltpu.VMEM((CHUNK, 128), jnp.float32),
            pltpu.VMEM((CHUNK, 128), jnp.float32),
            pltpu.SemaphoreType.DMA,
            pltpu.SemaphoreType.DMA,
            pltpu.SemaphoreType.DMA,
            pltpu.SemaphoreType.DMA,
        ],
    )
    def emb_kernel(idx_a, idx_r, wa, wr, out,
                   pv0, pv1, offv0, offv1, idxr0, idxr1,
                   stage0, stage1, gr0, gr1, sa0, sa1, sr0, sr1):
        wid = lax.axis_index("s") * NC + lax.axis_index("c")
        base = wid * BPW
        pv = (pv0, pv1)
        offv = (offv0, offv1)
        idxr = (idxr0, idxr1)
        stage = (stage0, stage1)
        gr = (gr0, gr1)
        sem_a = (sa0, sa1)
        sem_r = (sr0, sr1)

        def fetch(ci, b):
            start = base + ci * CHUNK
            pltpu.sync_copy(idx_a.at[pl.ds(start, CHUNK)], pv[b])
            pltpu.sync_copy(idx_r.at[pl.ds(start, CHUNK)], idxr[b])

            def map_body(g2, carry3):
                j0 = g2 * 16
                iv = pv[b][pl.ds(j0, 16)]
                hi = iv >= HALF
                pv[b][pl.ds(j0, 16)] = iv - jnp.where(hi, HALF, 0)
                offv[b][pl.ds(j0, 16)] = jnp.where(
                    hi & (iv < 2 * HALF), 64, 0)
                return carry3

            lax.fori_loop(0, CHUNK // 16, map_body, 0)
            pltpu.async_copy(wa.at[pv[b]], stage[b], sem_a[b])
            pltpu.async_copy(wr.at[idxr[b]], gr[b], sem_r[b])

        fetch(0, 0)

        def pair_body(g, carry):
            for b in (0, 1):
                ci = 2 * g + b

                @pl.when(ci + 1 < NCHUNK)
                def _prefetch():
                    fetch(ci + 1, 1 - b)

                pltpu.make_async_copy(
                    wa.at[pv[b]], stage[b], sem_a[b]).wait()
                pltpu.make_async_copy(
                    wr.at[idxr[b]], gr[b], sem_r[b]).wait()

                def node_body(i, carry2):
                    j0 = i * 16
                    ov = offv[b][pl.ds(j0, 16)]
                    for l in range(16):
                        j = j0 + l

                        @pl.when(ov[l] != 0)
                        def _shift_atom():
                            for k in range(4):
                                stage[b][j, pl.ds(16 * k, 16)] = (
                                    stage[b][j, pl.ds(64 + 16 * k, 16)])

                        for m in range(2):
                            stage[b][j, pl.ds(64 + 16 * m, 16)] = (
                                gr[b][j, pl.ds(16 * m, 16)])
                    return carry2

                lax.fori_loop(0, CHUNK // 16, node_body, 0)
                pltpu.sync_copy(stage[b],
                                out.at[pl.ds(base + ci * CHUNK, CHUNK)])
            return carry

        lax.fori_loop(0, NCHUNK // 2, pair_body, 0)

    return emb_kernel


_SC_EMB = _make_sc_kernel()


def kernel(atom_type, residue_type, charge, W_atom, W_res):
    idx_a = atom_type.reshape(-1).astype(jnp.int32)
    idx_r = residue_type.reshape(-1).astype(jnp.int32)
    pad = (0, NP - N)
    idx_a_p = jnp.pad(idx_a, pad)
    idx_r_p = jnp.pad(idx_r, pad)
    wt_res = jnp.pad(W_res.T, ((0, 0), (0, VRP - VR)))
    wr = _SC_RES(wt_res)
    wa = _relayout_atom(W_atom.T)
    out = _SC_EMB(idx_a_p, idx_r_p, wa, wr)
    return _finalize(out, charge.T).T
